# Initial kernel scaffold; baseline (speedup 1.0000x reference)
import jax
import jax.numpy as jnp
import numpy as np
from jax.experimental import pallas as pl
from jax.experimental.pallas import tpu as pltpu

NUM_CLASSES = 80
NUM_ANCHORS = 20000
BATCH = 8
MAX_TRUE = 100
POS_THRESH = 0.5
NEG_THRESH = 0.4
EPS = 1e-7

A_BLK = 2500
NA = NUM_ANCHORS // A_BLK


def _atan_pos(x):
    big = x > 2.414213562373095
    mid = x > 0.4142135623730951
    t = jnp.where(big, -1.0 / (x + EPS),
                  jnp.where(mid, (x - 1.0) / (x + 1.0), x))
    base = jnp.where(big, np.float32(np.pi / 2),
                     jnp.where(mid, np.float32(np.pi / 4), 0.0))
    z = t * t
    p = (((8.05374449538e-2 * z - 1.38776856032e-1) * z
          + 1.99777106478e-1) * z - 3.33329491539e-1) * z * t + t
    return base + p


def _body(yt_ref, btT_ref, conf_ref, logit_ref, bp_ref, anc_ref,
          out_ref, acc_ref):
    b = pl.program_id(0)
    ai = pl.program_id(1)

    @pl.when((b == 0) & (ai == 0))
    def _init():
        acc_ref[...] = jnp.zeros_like(acc_ref)

    anc = anc_ref[...]
    btT = btT_ref[0]
    ax1, ay1, ax2, ay2 = (anc[:, c:c + 1] for c in range(4))
    bx1, by1, bx2, by2 = (btT[c:c + 1, :] for c in range(4))

    ix1 = jnp.maximum(ax1, bx1)
    iy1 = jnp.maximum(ay1, by1)
    ix2 = jnp.minimum(ax2, bx2)
    iy2 = jnp.minimum(ay2, by2)
    inter = jnp.maximum(ix2 - ix1, 0.0) * jnp.maximum(iy2 - iy1, 0.0)
    area_a = jnp.maximum(ax2 - ax1, 0.0) * jnp.maximum(ay2 - ay1, 0.0)
    area_b = jnp.maximum(bx2 - bx1, 0.0) * jnp.maximum(by2 - by1, 0.0)
    iou = inter / (area_a + area_b - inter + EPS)
    valid = (bx1 > 0) | (by1 > 0) | (bx2 > 0) | (by2 > 0)
    iou = jnp.where(valid, iou, -1.0)

    max_iou = jnp.max(iou, axis=1, keepdims=True)
    pos = max_iou >= POS_THRESH
    pw = pos.astype(jnp.float32)
    tw = (pos | (max_iou < NEG_THRESH)).astype(jnp.float32)

    tidx = jax.lax.broadcasted_iota(jnp.int32, (1, MAX_TRUE), 1)
    m = iou == max_iou
    arg = jnp.min(jnp.where(m, tidx, MAX_TRUE), axis=1, keepdims=True)
    onehot = (tidx == arg).astype(jnp.float32)

    p = jnp.clip(conf_ref[0], EPS, 1.0 - EPS)
    bce = -(pw * jnp.log(p) + (1.0 - pw) * jnp.log(1.0 - p))
    score_part = jnp.sum(bce * tw)

    y_asn = jax.lax.dot_general(
        onehot, yt_ref[0], (((1,), (0,)), ((), ())),
        preferred_element_type=jnp.float32)
    q = jnp.clip(logit_ref[0], EPS, 1.0 - EPS)
    p_t = 1.0 - q + y_asn * (2.0 * q - 1.0)
    a_t = 0.75 - 0.5 * y_asn
    omp = 1.0 - p_t
    focal = a_t * omp * omp * (-jnp.log(p_t))
    class_part = jnp.sum(focal * pw)

    x1t = jnp.sum(onehot * bx1, axis=1, keepdims=True)
    y1t = jnp.sum(onehot * by1, axis=1, keepdims=True)
    x2t = jnp.sum(onehot * bx2, axis=1, keepdims=True)
    y2t = jnp.sum(onehot * by2, axis=1, keepdims=True)
    bp = bp_ref[0]
    x1p, y1p, x2p, y2p = (bp[:, c:c + 1] for c in range(4))
    wt = jnp.maximum(x2t - x1t, 0.0)
    ht = jnp.maximum(y2t - y1t, 0.0)
    wp = jnp.maximum(x2p - x1p, 0.0)
    hp = jnp.maximum(y2p - y1p, 0.0)
    binter = jnp.maximum(jnp.minimum(x2t, x2p) - jnp.maximum(x1t, x1p), 0.0) * \
             jnp.maximum(jnp.minimum(y2t, y2p) - jnp.maximum(y1t, y1p), 0.0)
    union = wt * ht + wp * hp - binter
    biou = binter / (union + EPS)
    cw = jnp.maximum(x2t, x2p) - jnp.minimum(x1t, x1p)
    ch = jnp.maximum(y2t, y2p) - jnp.minimum(y1t, y1p)
    c2 = cw * cw + ch * ch + EPS
    rho2 = ((x1t + x2t - x1p - x2p) ** 2 + (y1t + y2t - y1p - y2p) ** 2) / 4.0
    dat = _atan_pos(wt / (ht + EPS)) - _atan_pos(wp / (hp + EPS))
    v = np.float32(4.0 / (np.pi ** 2)) * dat * dat
    alpha = v / (1.0 - biou + v + EPS)
    cl = 1.0 - (biou - rho2 / c2 - alpha * v)
    bbox_part = jnp.sum(cl * pw)

    pos_cnt = jnp.sum(pw)

    lane = jax.lax.broadcasted_iota(jnp.int32, (1, 128), 1)
    vec = (jnp.where(lane == 0, score_part, 0.0)
           + jnp.where(lane == 1, class_part, 0.0)
           + jnp.where(lane == 2, bbox_part, 0.0)
           + jnp.where(lane == 3, pos_cnt, 0.0))
    acc_ref[pl.ds(b, 1), :] += vec

    @pl.when((b == BATCH - 1) & (ai == NA - 1))
    def _fin():
        acc = acc_ref[...]
        avg = jnp.sum(jnp.maximum(acc[:, 3:4], 1.0))
        sums = jnp.sum(acc, axis=0, keepdims=True)
        losses = sums / avg
        bad = jnp.isnan(losses) | jnp.isinf(losses)
        out_ref[...] = jnp.where(bad, 0.0, losses)


def kernel(y_true, bbox_true, conf_pred, logit_pred, bbox_pred, anchors):
    btT = jnp.transpose(bbox_true, (0, 2, 1))
    out = pl.pallas_call(
        _body,
        grid=(BATCH, NA),
        in_specs=[
            pl.BlockSpec((1, MAX_TRUE, NUM_CLASSES), lambda b, ai: (b, 0, 0)),
            pl.BlockSpec((1, 4, MAX_TRUE), lambda b, ai: (b, 0, 0)),
            pl.BlockSpec((1, A_BLK, 1), lambda b, ai: (b, ai, 0)),
            pl.BlockSpec((1, A_BLK, NUM_CLASSES), lambda b, ai: (b, ai, 0)),
            pl.BlockSpec((1, A_BLK, 4), lambda b, ai: (b, ai, 0)),
            pl.BlockSpec((A_BLK, 4), lambda b, ai: (ai, 0)),
        ],
        out_specs=pl.BlockSpec((1, 128), lambda b, ai: (0, 0)),
        out_shape=jax.ShapeDtypeStruct((1, 128), jnp.float32),
        scratch_shapes=[pltpu.VMEM((8, 128), jnp.float32)],
        compiler_params=pltpu.CompilerParams(
            dimension_semantics=("arbitrary", "arbitrary")),
    )(y_true, btT, conf_pred, logit_pred, bbox_pred, anchors)
    return out[0, :3]


# speedup vs baseline: 2.2518x; 2.2518x over previous
import jax
import jax.numpy as jnp
import numpy as np
from jax.experimental import pallas as pl
from jax.experimental.pallas import tpu as pltpu

NUM_CLASSES = 80
NUM_ANCHORS = 20000
BATCH = 8
MAX_TRUE = 100
POS_THRESH = 0.5
NEG_THRESH = 0.4
EPS = 1e-7

A_BLK = 2000
NA = NUM_ANCHORS // A_BLK


def _atan_pos(x):
    big = x > 2.414213562373095
    mid = x > 0.4142135623730951
    t = jnp.where(big, -1.0 / (x + EPS),
                  jnp.where(mid, (x - 1.0) / (x + 1.0), x))
    base = jnp.where(big, np.float32(np.pi / 2),
                     jnp.where(mid, np.float32(np.pi / 4), 0.0))
    z = t * t
    p = (((8.05374449538e-2 * z - 1.38776856032e-1) * z
          + 1.99777106478e-1) * z - 3.33329491539e-1) * z * t + t
    return base + p


def _body(yt_ref, btT_ref, conf_ref, logit_ref, bp_ref, anc_ref,
          out_ref, acc_ref):
    b = pl.program_id(0)
    ai = pl.program_id(1)

    @pl.when((b == 0) & (ai == 0))
    def _init():
        acc_ref[...] = jnp.zeros_like(acc_ref)

    anc = anc_ref[...]
    btT = btT_ref[0]
    ax1, ay1, ax2, ay2 = (anc[:, c:c + 1] for c in range(4))
    bx1, by1, bx2, by2 = (btT[c:c + 1, :] for c in range(4))

    ix1 = jnp.maximum(ax1, bx1)
    iy1 = jnp.maximum(ay1, by1)
    ix2 = jnp.minimum(ax2, bx2)
    iy2 = jnp.minimum(ay2, by2)
    inter = jnp.maximum(ix2 - ix1, 0.0) * jnp.maximum(iy2 - iy1, 0.0)
    area_a = jnp.maximum(ax2 - ax1, 0.0) * jnp.maximum(ay2 - ay1, 0.0)
    area_b = jnp.maximum(bx2 - bx1, 0.0) * jnp.maximum(by2 - by1, 0.0)
    iou = inter / (area_a + area_b - inter + EPS)
    valid = (bx1 > 0) | (by1 > 0) | (bx2 > 0) | (by2 > 0)
    iou = jnp.where(valid, iou, -1.0)

    max_iou = jnp.max(iou, axis=1, keepdims=True)
    pos = max_iou >= POS_THRESH
    pw = pos.astype(jnp.float32)
    tw = (pos | (max_iou < NEG_THRESH)).astype(jnp.float32)

    tidx = jax.lax.broadcasted_iota(jnp.int32, (1, MAX_TRUE), 1)
    m = iou == max_iou
    arg = jnp.min(jnp.where(m, tidx, MAX_TRUE), axis=1, keepdims=True)
    onehot = (tidx == arg).astype(jnp.float32)

    p = jnp.clip(conf_ref[0], EPS, 1.0 - EPS)
    bce = -(pw * jnp.log(p) + (1.0 - pw) * jnp.log(1.0 - p))
    score_part = jnp.sum(bce * tw)

    y_asn = jax.lax.dot_general(
        onehot, yt_ref[0], (((1,), (0,)), ((), ())),
        preferred_element_type=jnp.float32)
    q = jnp.clip(logit_ref[0], EPS, 1.0 - EPS)
    p_t = 1.0 - q + y_asn * (2.0 * q - 1.0)
    a_t = 0.75 - 0.5 * y_asn
    omp = 1.0 - p_t
    focal = a_t * omp * omp * (-jnp.log(p_t))
    class_part = jnp.sum(focal * pw)

    x1t = jnp.sum(onehot * bx1, axis=1, keepdims=True)
    y1t = jnp.sum(onehot * by1, axis=1, keepdims=True)
    x2t = jnp.sum(onehot * bx2, axis=1, keepdims=True)
    y2t = jnp.sum(onehot * by2, axis=1, keepdims=True)
    bp = bp_ref[0]
    x1p, y1p, x2p, y2p = (bp[:, c:c + 1] for c in range(4))
    wt = jnp.maximum(x2t - x1t, 0.0)
    ht = jnp.maximum(y2t - y1t, 0.0)
    wp = jnp.maximum(x2p - x1p, 0.0)
    hp = jnp.maximum(y2p - y1p, 0.0)
    binter = jnp.maximum(jnp.minimum(x2t, x2p) - jnp.maximum(x1t, x1p), 0.0) * \
             jnp.maximum(jnp.minimum(y2t, y2p) - jnp.maximum(y1t, y1p), 0.0)
    union = wt * ht + wp * hp - binter
    biou = binter / (union + EPS)
    cw = jnp.maximum(x2t, x2p) - jnp.minimum(x1t, x1p)
    ch = jnp.maximum(y2t, y2p) - jnp.minimum(y1t, y1p)
    c2 = cw * cw + ch * ch + EPS
    rho2 = ((x1t + x2t - x1p - x2p) ** 2 + (y1t + y2t - y1p - y2p) ** 2) / 4.0
    dat = _atan_pos(wt / (ht + EPS)) - _atan_pos(wp / (hp + EPS))
    v = np.float32(4.0 / (np.pi ** 2)) * dat * dat
    alpha = v / (1.0 - biou + v + EPS)
    cl = 1.0 - (biou - rho2 / c2 - alpha * v)
    bbox_part = jnp.sum(cl * pw)

    pos_cnt = jnp.sum(pw)

    lane = jax.lax.broadcasted_iota(jnp.int32, (1, 128), 1)
    vec = (jnp.where(lane == 0, score_part, 0.0)
           + jnp.where(lane == 1, class_part, 0.0)
           + jnp.where(lane == 2, bbox_part, 0.0)
           + jnp.where(lane == 3, pos_cnt, 0.0))
    acc_ref[pl.ds(b, 1), :] += vec

    @pl.when((b == BATCH - 1) & (ai == NA - 1))
    def _fin():
        acc = acc_ref[...]
        avg = jnp.sum(jnp.maximum(acc[:, 3:4], 1.0))
        sums = jnp.sum(acc, axis=0, keepdims=True)
        losses = sums / avg
        bad = jnp.isnan(losses) | jnp.isinf(losses)
        out_ref[...] = jnp.where(bad, 0.0, losses)


def kernel(y_true, bbox_true, conf_pred, logit_pred, bbox_pred, anchors):
    btT = jnp.transpose(bbox_true, (0, 2, 1))
    out = pl.pallas_call(
        _body,
        grid=(BATCH, NA),
        in_specs=[
            pl.BlockSpec((1, MAX_TRUE, NUM_CLASSES), lambda b, ai: (b, 0, 0)),
            pl.BlockSpec((1, 4, MAX_TRUE), lambda b, ai: (b, 0, 0)),
            pl.BlockSpec((1, A_BLK, 1), lambda b, ai: (b, ai, 0)),
            pl.BlockSpec((1, A_BLK, NUM_CLASSES), lambda b, ai: (b, ai, 0)),
            pl.BlockSpec((1, A_BLK, 4), lambda b, ai: (b, ai, 0)),
            pl.BlockSpec((A_BLK, 4), lambda b, ai: (ai, 0)),
        ],
        out_specs=pl.BlockSpec((1, 128), lambda b, ai: (0, 0)),
        out_shape=jax.ShapeDtypeStruct((1, 128), jnp.float32),
        scratch_shapes=[pltpu.VMEM((8, 128), jnp.float32)],
        compiler_params=pltpu.CompilerParams(
            dimension_semantics=("arbitrary", "arbitrary")),
    )(y_true, btT, conf_pred, logit_pred, bbox_pred, anchors)
    return out[0, :3]


# speedup vs baseline: 11.7765x; 5.2299x over previous
import jax
import jax.numpy as jnp
import numpy as np
from jax.experimental import pallas as pl
from jax.experimental.pallas import tpu as pltpu

NUM_CLASSES = 80
NUM_ANCHORS = 20000
A_PAD = 20480
BATCH = 8
MAX_TRUE = 100
POS_THRESH = 0.5
NEG_THRESH = 0.4
EPS = 1e-7

A_BLK = 2048
NA = A_PAD // A_BLK


def _atan_pos(x):
    big = x > 2.414213562373095
    mid = x > 0.4142135623730951
    t = jnp.where(big, -1.0 / (x + EPS),
                  jnp.where(mid, (x - 1.0) / (x + 1.0), x))
    base = jnp.where(big, np.float32(np.pi / 2),
                     jnp.where(mid, np.float32(np.pi / 4), 0.0))
    z = t * t
    p = (((8.05374449538e-2 * z - 1.38776856032e-1) * z
          + 1.99777106478e-1) * z - 3.33329491539e-1) * z * t + t
    return base + p


def _body(yb_ref, bt_ref, conf_ref, logit_ref, bp_ref, anc_ref,
          out_ref, acc_ref):
    b = pl.program_id(0)
    ai = pl.program_id(1)

    @pl.when((b == 0) & (ai == 0))
    def _init():
        acc_ref[...] = jnp.zeros_like(acc_ref)

    anc = anc_ref[...]
    bt = bt_ref[0]
    ax1, ay1, ax2, ay2 = (anc[c:c + 1, :] for c in range(4))
    bx1, by1, bx2, by2 = (bt[:, c:c + 1] for c in range(4))

    ix1 = jnp.maximum(ax1, bx1)
    iy1 = jnp.maximum(ay1, by1)
    ix2 = jnp.minimum(ax2, bx2)
    iy2 = jnp.minimum(ay2, by2)
    inter = jnp.maximum(ix2 - ix1, 0.0) * jnp.maximum(iy2 - iy1, 0.0)
    area_a = jnp.maximum(ax2 - ax1, 0.0) * jnp.maximum(ay2 - ay1, 0.0)
    area_b = jnp.maximum(bx2 - bx1, 0.0) * jnp.maximum(by2 - by1, 0.0)
    iou = inter / (area_a + area_b - inter + EPS)
    valid = (bx1 > 0) | (by1 > 0) | (bx2 > 0) | (by2 > 0)
    iou = jnp.where(valid, iou, -1.0)

    max_iou = jnp.max(iou, axis=0, keepdims=True)
    lane = jax.lax.broadcasted_iota(jnp.int32, (1, A_BLK), 1)
    amask = ai * A_BLK + lane < NUM_ANCHORS
    pos = (max_iou >= POS_THRESH) & amask
    pw = pos.astype(jnp.float32)
    tw = ((max_iou >= POS_THRESH) | (max_iou < NEG_THRESH)).astype(
        jnp.float32) * amask.astype(jnp.float32)

    tidx = jax.lax.broadcasted_iota(jnp.int32, (MAX_TRUE, 1), 0)
    m = iou == max_iou
    arg = jnp.min(jnp.where(m, tidx, MAX_TRUE), axis=0, keepdims=True)
    onehot = (tidx == arg).astype(jnp.float32)

    asn = jax.lax.dot_general(
        yb_ref[0], onehot, (((1,), (0,)), ((), ())),
        preferred_element_type=jnp.float32)

    p = jnp.clip(conf_ref[0], EPS, 1.0 - EPS)
    bce = -(pw * jnp.log(p) + (1.0 - pw) * jnp.log(1.0 - p))
    score_part = jnp.sum(bce * tw)

    y_asn = asn[:NUM_CLASSES, :]
    q = jnp.clip(logit_ref[0], EPS, 1.0 - EPS)
    p_t = 1.0 - q + y_asn * (2.0 * q - 1.0)
    a_t = 0.75 - 0.5 * y_asn
    omp = 1.0 - p_t
    focal = a_t * omp * omp * (-jnp.log(p_t))
    class_part = jnp.sum(focal * pw)

    x1t = asn[NUM_CLASSES + 0:NUM_CLASSES + 1, :]
    y1t = asn[NUM_CLASSES + 1:NUM_CLASSES + 2, :]
    x2t = asn[NUM_CLASSES + 2:NUM_CLASSES + 3, :]
    y2t = asn[NUM_CLASSES + 3:NUM_CLASSES + 4, :]
    bp = bp_ref[0]
    x1p, y1p, x2p, y2p = (bp[c:c + 1, :] for c in range(4))
    wt = jnp.maximum(x2t - x1t, 0.0)
    ht = jnp.maximum(y2t - y1t, 0.0)
    wp = jnp.maximum(x2p - x1p, 0.0)
    hp = jnp.maximum(y2p - y1p, 0.0)
    binter = jnp.maximum(jnp.minimum(x2t, x2p) - jnp.maximum(x1t, x1p), 0.0) * \
             jnp.maximum(jnp.minimum(y2t, y2p) - jnp.maximum(y1t, y1p), 0.0)
    union = wt * ht + wp * hp - binter
    biou = binter / (union + EPS)
    cw = jnp.maximum(x2t, x2p) - jnp.minimum(x1t, x1p)
    ch = jnp.maximum(y2t, y2p) - jnp.minimum(y1t, y1p)
    c2 = cw * cw + ch * ch + EPS
    rho2 = ((x1t + x2t - x1p - x2p) ** 2 + (y1t + y2t - y1p - y2p) ** 2) / 4.0
    dat = _atan_pos(wt / (ht + EPS)) - _atan_pos(wp / (hp + EPS))
    v = np.float32(4.0 / (np.pi ** 2)) * dat * dat
    alpha = v / (1.0 - biou + v + EPS)
    cl = 1.0 - (biou - rho2 / c2 - alpha * v)
    bbox_part = jnp.sum(cl * pw)

    pos_cnt = jnp.sum(pw)

    lidx = jax.lax.broadcasted_iota(jnp.int32, (1, 128), 1)
    vec = (jnp.where(lidx == 0, score_part, 0.0)
           + jnp.where(lidx == 1, class_part, 0.0)
           + jnp.where(lidx == 2, bbox_part, 0.0)
           + jnp.where(lidx == 3, pos_cnt, 0.0))
    acc_ref[pl.ds(b, 1), :] += vec

    @pl.when((b == BATCH - 1) & (ai == NA - 1))
    def _fin():
        acc = acc_ref[...]
        avg = jnp.sum(jnp.maximum(acc[:, 3:4], 1.0))
        sums = jnp.sum(acc, axis=0, keepdims=True)
        losses = sums / avg
        bad = jnp.isnan(losses) | jnp.isinf(losses)
        out_ref[...] = jnp.where(bad, 0.0, losses)


def kernel(y_true, bbox_true, conf_pred, logit_pred, bbox_pred, anchors):
    pad = A_PAD - NUM_ANCHORS
    yb = jnp.concatenate([jnp.transpose(y_true, (0, 2, 1)),
                          jnp.transpose(bbox_true, (0, 2, 1))], axis=1)
    confT = jnp.pad(jnp.reshape(conf_pred, (BATCH, 1, NUM_ANCHORS)),
                    ((0, 0), (0, 0), (0, pad)))
    logitT = jnp.pad(jnp.transpose(logit_pred, (0, 2, 1)),
                     ((0, 0), (0, 0), (0, pad)))
    bpT = jnp.pad(jnp.transpose(bbox_pred, (0, 2, 1)),
                  ((0, 0), (0, 0), (0, pad)))
    ancT = jnp.pad(jnp.transpose(anchors, (1, 0)), ((0, 0), (0, pad)))
    out = pl.pallas_call(
        _body,
        grid=(BATCH, NA),
        in_specs=[
            pl.BlockSpec((1, NUM_CLASSES + 4, MAX_TRUE), lambda b, ai: (b, 0, 0)),
            pl.BlockSpec((1, MAX_TRUE, 4), lambda b, ai: (b, 0, 0)),
            pl.BlockSpec((1, 1, A_BLK), lambda b, ai: (b, 0, ai)),
            pl.BlockSpec((1, NUM_CLASSES, A_BLK), lambda b, ai: (b, 0, ai)),
            pl.BlockSpec((1, 4, A_BLK), lambda b, ai: (b, 0, ai)),
            pl.BlockSpec((4, A_BLK), lambda b, ai: (0, ai)),
        ],
        out_specs=pl.BlockSpec((1, 128), lambda b, ai: (0, 0)),
        out_shape=jax.ShapeDtypeStruct((1, 128), jnp.float32),
        scratch_shapes=[pltpu.VMEM((8, 128), jnp.float32)],
        compiler_params=pltpu.CompilerParams(
            dimension_semantics=("arbitrary", "arbitrary")),
    )(yb, bbox_true, confT, logitT, bpT, ancT)
    return out[0, :3]
